# Initial kernel scaffold; baseline (speedup 1.0000x reference)
import jax
import jax.numpy as jnp
from jax.experimental import pallas as pl


def kernel(center, contexts_and_negatives, central_table, context_table):
    raise NotImplementedError("write your pallas kernel here")


# speedup vs baseline: 1.2654x; 1.2654x over previous
import functools

import jax
import jax.numpy as jnp
from jax import lax
from jax.experimental import pallas as pl
from jax.experimental.pallas import tpu as pltpu
from jax.experimental.pallas import tpu_sc as plsc

_B = 16384
_L = 50
_D = 64

_NC = 2
_NS = 16
_NW = _NC * _NS
_BPW = _B // _NW
_CB = 4
_CHUNK = _CB * _L
_NCHUNK = _BPW // _CB
_GSUB = 40
_TP = 17


def _make_sc_kernel():
    mesh = plsc.VectorSubcoreMesh(core_axis_name="c", subcore_axis_name="s")

    @functools.partial(
        pl.kernel,
        mesh=mesh,
        compiler_params=pltpu.CompilerParams(needs_layout_passes=False,
                                             use_tc_tiling_on_sc=False),
        out_type=jax.ShapeDtypeStruct((_B * _L,), jnp.float32),
        scratch_types=[
            pltpu.VMEM((_BPW,), jnp.int32),
            pltpu.VMEM((_BPW, _D), jnp.float32),
            pltpu.VMEM((_CHUNK,), jnp.int32),
            pltpu.VMEM((_CHUNK, _D), jnp.float32),
            pltpu.VMEM((16 * _TP,), jnp.float32),
            pltpu.VMEM((_CHUNK,), jnp.float32),
            pltpu.SemaphoreType.DMA,
        ],
    )
    def sc_kernel(center_hbm, ctx_hbm, cen_tab, ctx_tab, out_hbm,
                  idx_c, v_rows, idx_v, u_rows, tbuf, res, sem):
        wid = lax.axis_index("s") * _NC + lax.axis_index("c")
        iota = lax.iota(jnp.int32, 16)

        pltpu.sync_copy(center_hbm.at[pl.ds(wid * _BPW, _BPW)], idx_c)
        cps = [
            pltpu.async_copy(cen_tab.at[idx_c.at[pl.ds(k * 128, 128)]],
                             v_rows.at[pl.ds(k * 128, 128), :], sem)
            for k in range(_BPW // 128)
        ]
        for cp in cps:
            cp.wait()

        base_w = wid * (_BPW * _L)

        def chunk_body(g, carry):
            base = base_w + g * _CHUNK
            pltpu.sync_copy(ctx_hbm.at[pl.ds(base, _CHUNK)], idx_v)
            gs = [
                pltpu.async_copy(ctx_tab.at[idx_v.at[pl.ds(j * _GSUB, _GSUB)]],
                                 u_rows.at[pl.ds(j * _GSUB, _GSUB), :], sem)
                for j in range(_CHUNK // _GSUB)
            ]
            for cp in gs:
                cp.wait()

            gb0 = g * _CB
            for b in range(_CB):
                vv = [v_rows[gb0 + b, pl.ds(dc * 16, 16)] for dc in range(4)]
                for l0, nl in ((0, 16), (16, 16), (32, 16), (48, 2)):
                    for li in range(nl):
                        row = b * _L + l0 + li
                        p = u_rows[row, pl.ds(0, 16)] * vv[0]
                        for dc in range(1, 4):
                            p = p + u_rows[row, pl.ds(dc * 16, 16)] * vv[dc]
                        plsc.store_scatter(tbuf, [iota + li * _TP], p)
                    acc = plsc.load_gather(tbuf, [iota * _TP])
                    for jj in range(1, 16):
                        acc = acc + plsc.load_gather(tbuf, [iota * _TP + jj])
                    oidx = iota + (b * _L + l0)
                    if nl == 16:
                        plsc.store_scatter(res, [oidx], acc)
                    else:
                        m = iota < nl
                        plsc.store_scatter(res, [jnp.where(m, oidx, 0)], acc,
                                           mask=m)
            pltpu.sync_copy(res, out_hbm.at[pl.ds(base, _CHUNK)])
            return carry

        lax.fori_loop(0, _NCHUNK, chunk_body, 0)

    return sc_kernel


@functools.cache
def _sc_kernel_cached():
    return _make_sc_kernel()


def kernel(center, contexts_and_negatives, central_table, context_table):
    out_flat = _sc_kernel_cached()(center.reshape(-1),
                          contexts_and_negatives.reshape(-1),
                          central_table, context_table)
    return out_flat.reshape(_B, 1, _L)
